# Initial kernel scaffold; baseline (speedup 1.0000x reference)
import jax
import jax.numpy as jnp
from jax.experimental import pallas as pl


def kernel(x, edge_index, W_fc, attn_l, attn_r, bias_gat, W44, b44):
    raise NotImplementedError("write your pallas kernel here")


# speedup vs baseline: 32.1590x; 32.1590x over previous
import jax
import jax.numpy as jnp
from jax import lax
from jax.experimental import pallas as pl
from jax.experimental.pallas import tpu as pltpu
from jax.experimental.pallas import tpu_sc as plsc

N = 10000
E = 320000
D = 128
H = 8
OUT = 32
HC = H // 2
FW = HC * OUT
EB = 80
NSUB = 16
EPT = E // NSUB
NBATCH = EPT // EB
RPT = 640


def _stripe(s, fn):
    @pl.when(s < NSUB - 1)
    def _():
        fn(pl.multiple_of(s * RPT, 8), RPT)

    @pl.when(s == NSUB - 1)
    def _():
        fn((NSUB - 1) * RPT, N - (NSUB - 1) * RPT)


def _sc_edge(src_hbm, dst_hbm, f_hbm, el_hbm, er_hbm, zf_hbm, zd_hbm,
             u_out, d_out,
             srcb, dstb, idxb, didxb, featb, elsb, erdb, wpad,
             u_sp, d_sp, sem):
    c = lax.axis_index("c")
    s = lax.axis_index("s")

    def zinit(r0, nr):
        pltpu.sync_copy(zf_hbm.at[pl.ds(r0, nr)], u_sp.at[pl.ds(r0, nr)])
        pltpu.sync_copy(zd_hbm.at[pl.ds(r0, nr)], d_sp.at[pl.ds(r0, nr)])

    _stripe(s, zinit)
    plsc.subcore_barrier()

    ebase = s * EPT
    cN = c * N

    def batch(j, carry):
        base = pl.multiple_of(ebase + j * EB, 8)
        pltpu.sync_copy(src_hbm.at[pl.ds(base, EB)], srcb)
        pltpu.sync_copy(dst_hbm.at[pl.ds(base, EB)], dstb)

        def mk(g, carry2):
            idxb[pl.ds(g * 16, 16)] = srcb[pl.ds(g * 16, 16)] + cN
            didxb[pl.ds(g * 16, 16)] = dstb[pl.ds(g * 16, 16)] + cN
            return carry2

        lax.fori_loop(0, EB // 16, mk, 0)
        cpf = pltpu.async_copy(f_hbm.at[idxb], featb, sem)
        cpl = pltpu.async_copy(el_hbm.at[idxb], elsb, sem)
        cpr = pltpu.async_copy(er_hbm.at[didxb], erdb, sem)
        cpl.wait()
        cpr.wait()
        cpf.wait()

        def edge(i, carry2):
            e = elsb[i, :] + erdb[i, :]
            e = jnp.where(e >= 0.0, e, 0.2 * e)
            w = jnp.exp(e)
            wpad[i, :] = w
            for hh in range(HC):
                wv = jnp.broadcast_to(w[hh], (16,))
                for k in range(2):
                    col = hh * OUT + k * 16
                    featb[i, pl.ds(col, 16)] = featb[i, pl.ds(col, 16)] * wv
            return carry2

        lax.fori_loop(0, EB, edge, 0)
        pltpu.sync_copy(featb, u_sp.at[dstb], add=True)
        pltpu.sync_copy(wpad, d_sp.at[dstb], add=True)
        return carry

    lax.fori_loop(0, NBATCH, batch, 0)
    plsc.subcore_barrier()

    def wout(r0, nr):
        pltpu.sync_copy(u_sp.at[pl.ds(r0, nr)], u_out.at[c, pl.ds(r0, nr)])
        pltpu.sync_copy(d_sp.at[pl.ds(r0, nr)], d_out.at[c, pl.ds(r0, nr)])

    _stripe(s, wout)


def _sc_call(src, dst, fT, elP, erP, zf, zd):
    mesh = plsc.VectorSubcoreMesh(core_axis_name="c", subcore_axis_name="s")
    return pl.kernel(
        _sc_edge,
        out_type=[jax.ShapeDtypeStruct((2, N, FW), jnp.float32),
                  jax.ShapeDtypeStruct((2, N, 16), jnp.float32)],
        mesh=mesh,
        scratch_types=[
            pltpu.VMEM((EB,), jnp.int32),
            pltpu.VMEM((EB,), jnp.int32),
            pltpu.VMEM((EB,), jnp.int32),
            pltpu.VMEM((EB,), jnp.int32),
            pltpu.VMEM((EB, FW), jnp.float32),
            pltpu.VMEM((EB, 16), jnp.float32),
            pltpu.VMEM((EB, 16), jnp.float32),
            pltpu.VMEM((EB, 16), jnp.float32),
            pltpu.VMEM_SHARED((N, FW), jnp.float32),
            pltpu.VMEM_SHARED((N, 16), jnp.float32),
            pltpu.SemaphoreType.DMA,
        ],
        compiler_params=pltpu.CompilerParams(
            needs_layout_passes=False, use_tc_tiling_on_sc=False),
    )(src, dst, fT, elP, erP, zf, zd)


RB = 1000


def _prologue(x_ref, w_ref, al_ref, ar_ref, f_ref, el_ref, er_ref):
    c = pl.program_id(1)
    fc = jnp.dot(x_ref[...], w_ref[...], preferred_element_type=jnp.float32)
    f_ref[0] = fc
    pel = jnp.dot(fc, al_ref[...], preferred_element_type=jnp.float32)
    per = jnp.dot(fc, ar_ref[...], preferred_element_type=jnp.float32)

    @pl.when(c == 0)
    def _():
        el_ref[...] = pel
        er_ref[...] = per

    @pl.when(c == 1)
    def _():
        el_ref[...] = el_ref[...] + pel
        er_ref[...] = er_ref[...] + per


def _epilogue(u_ref, d_ref, sel_ref, bias_ref, w44_ref, b44_ref, o_ref):
    acc = jnp.zeros((RB, OUT), jnp.float32)
    for c in range(2):
        uc = u_ref[c]
        dinv = 1.0 / jnp.maximum(d_ref[c][:, 0:HC], 1e-9)
        dfull = jnp.dot(dinv, sel_ref[...],
                        preferred_element_type=jnp.float32)
        r = uc * dfull + bias_ref[c][None, :]
        r = jnp.where(r > 0.0, r, jnp.exp(r) - 1.0)
        acc = acc + jnp.dot(r, w44_ref[c], preferred_element_type=jnp.float32)
    o_ref[...] = acc + b44_ref[...]


def kernel(x, edge_index, W_fc, attn_l, attn_r, bias_gat, W44, b44):
    src = edge_index[0]
    dst = edge_index[1]

    rows = jnp.arange(H * OUT, dtype=jnp.int32)
    Al = jnp.zeros((H * OUT, H), jnp.float32).at[rows, rows // OUT].set(
        attn_l.reshape(-1))
    Ar = jnp.zeros((H * OUT, H), jnp.float32).at[rows, rows // OUT].set(
        attn_r.reshape(-1))

    fT, el, er = pl.pallas_call(
        _prologue,
        grid=(N // RB, 2),
        in_specs=[
            pl.BlockSpec((RB, D), lambda i, c: (i, 0)),
            pl.BlockSpec((D, FW), lambda i, c: (0, c)),
            pl.BlockSpec((FW, H), lambda i, c: (c, 0)),
            pl.BlockSpec((FW, H), lambda i, c: (c, 0)),
        ],
        out_specs=[
            pl.BlockSpec((1, RB, FW), lambda i, c: (c, i, 0)),
            pl.BlockSpec((RB, H), lambda i, c: (i, 0)),
            pl.BlockSpec((RB, H), lambda i, c: (i, 0)),
        ],
        out_shape=[
            jax.ShapeDtypeStruct((2, N, FW), jnp.float32),
            jax.ShapeDtypeStruct((N, H), jnp.float32),
            jax.ShapeDtypeStruct((N, H), jnp.float32),
        ],
    )(x, W_fc, Al, Ar)

    def pad16(t):
        return jnp.pad(
            t.T.reshape(2, HC, N).transpose(0, 2, 1),
            ((0, 0), (0, 0), (0, 16 - HC))).reshape(2 * N, 16)

    elP = pad16(el)
    erP = pad16(er)

    zf = jnp.zeros((N, FW), jnp.float32)
    zd = jnp.zeros((N, 16), jnp.float32)

    U, den = _sc_call(src, dst, fT.reshape(2 * N, FW), elP, erP, zf, zd)

    sel = jnp.kron(jnp.eye(HC, dtype=jnp.float32),
                   jnp.ones((1, OUT), jnp.float32))

    out = pl.pallas_call(
        _epilogue,
        grid=(N // RB,),
        in_specs=[
            pl.BlockSpec((2, RB, FW), lambda i: (0, i, 0)),
            pl.BlockSpec((2, RB, 16), lambda i: (0, i, 0)),
            pl.BlockSpec((HC, FW), lambda i: (0, 0)),
            pl.BlockSpec((2, FW), lambda i: (0, 0)),
            pl.BlockSpec((2, FW, OUT), lambda i: (0, 0, 0)),
            pl.BlockSpec((1, OUT), lambda i: (0, 0)),
        ],
        out_specs=pl.BlockSpec((RB, OUT), lambda i: (i, 0)),
        out_shape=jax.ShapeDtypeStruct((N, OUT), jnp.float32),
    )(U, den, sel, bias_gat.reshape(2, FW), W44.reshape(2, FW, OUT),
      b44.reshape(1, OUT))
    return out


# speedup vs baseline: 52.5107x; 1.6328x over previous
import jax
import jax.numpy as jnp
from jax import lax
from jax.experimental import pallas as pl
from jax.experimental.pallas import tpu as pltpu
from jax.experimental.pallas import tpu_sc as plsc

N = 10000
E = 320000
D = 128
H = 8
OUT = 32
HC = H // 2
FW = HC * OUT
EB = 80
NSUB = 16
EPT = E // NSUB
NBATCH = EPT // EB
RPT = 640


def _stripe(s, fn):
    @pl.when(s < NSUB - 1)
    def _():
        fn(pl.multiple_of(s * RPT, 8), RPT)

    @pl.when(s == NSUB - 1)
    def _():
        fn((NSUB - 1) * RPT, N - (NSUB - 1) * RPT)


def _sc_edge(src_hbm, dst_hbm, f_hbm, el_hbm, er_hbm, zf_hbm, zd_hbm,
             u_out, d_out,
             srcb, dstb, idxb, didxb, featb, elsb, erdb, wpad,
             u_sp, d_sp, gsem0, gsem1, ssem0, ssem1):
    c = lax.axis_index("c")
    s = lax.axis_index("s")
    gsem = (gsem0, gsem1)
    ssem = (ssem0, ssem1)

    def zinit(r0, nr):
        pltpu.sync_copy(zf_hbm.at[pl.ds(r0, nr)], u_sp.at[pl.ds(r0, nr)])
        pltpu.sync_copy(zd_hbm.at[pl.ds(r0, nr)], d_sp.at[pl.ds(r0, nr)])

    _stripe(s, zinit)
    plsc.subcore_barrier()

    ebase = s * EPT
    cN = c * N

    def fire(j, b):
        base = pl.multiple_of(ebase + j * EB, 8)
        pltpu.sync_copy(src_hbm.at[pl.ds(base, EB)], srcb.at[b])
        pltpu.sync_copy(dst_hbm.at[pl.ds(base, EB)], dstb.at[b])

        def mk(g, carry2):
            idxb[b, pl.ds(g * 16, 16)] = srcb[b, pl.ds(g * 16, 16)] + cN
            didxb[b, pl.ds(g * 16, 16)] = dstb[b, pl.ds(g * 16, 16)] + cN
            return carry2

        lax.fori_loop(0, EB // 16, mk, 0)
        pltpu.async_copy(f_hbm.at[idxb.at[b]], featb.at[b], gsem[b])
        pltpu.async_copy(el_hbm.at[idxb.at[b]], elsb.at[b], gsem[b])
        pltpu.async_copy(er_hbm.at[didxb.at[b]], erdb.at[b], gsem[b])

    def wait_gathers(b):
        pltpu.make_async_copy(f_hbm.at[idxb.at[b]], featb.at[b], gsem[b]).wait()
        pltpu.make_async_copy(el_hbm.at[idxb.at[b]], elsb.at[b], gsem[b]).wait()
        pltpu.make_async_copy(er_hbm.at[didxb.at[b]], erdb.at[b],
                              gsem[b]).wait()

    def fire_scatter(b):
        pltpu.async_copy(featb.at[b], u_sp.at[dstb.at[b]], ssem[b], add=True)
        pltpu.async_copy(wpad.at[b], d_sp.at[dstb.at[b]], ssem[b], add=True)

    def wait_scatter(b):
        pltpu.make_async_copy(featb.at[b], u_sp.at[dstb.at[b]],
                              ssem[b]).wait()
        pltpu.make_async_copy(wpad.at[b], d_sp.at[dstb.at[b]], ssem[b]).wait()

    def proc(j, b, first, last):
        b1 = 1 - b
        wait_gathers(b)

        def wphase(i, carry2):
            e = elsb[b, i, :] + erdb[b, i, :]
            e = jnp.where(e >= 0.0, e, 0.2 * e)
            wpad[b, i, :] = jnp.exp(e)
            return carry2

        lax.fori_loop(0, EB, wphase, 0)

        if first is None:
            wait_scatter(b1)
            fire(j + 1, b1)
        elif first:
            fire(j + 1, b1)
        else:
            wait_scatter(b1)

        def scale(i, carry2):
            w = wpad[b, i, :]
            for hh in range(HC):
                wv = jnp.broadcast_to(w[hh], (16,))
                for k in range(2):
                    col = hh * OUT + k * 16
                    featb[b, i, pl.ds(col, 16)] = (
                        featb[b, i, pl.ds(col, 16)] * wv)
            return carry2

        lax.fori_loop(0, EB, scale, 0, unroll=2)
        fire_scatter(b)

    fire(0, 0)
    proc(0, 0, True, False)

    def pair(g, carry):
        j0 = 2 * g + 1
        proc(j0, 1, None, False)
        proc(j0 + 1, 0, None, False)
        return carry

    lax.fori_loop(0, (NBATCH - 2) // 2, pair, 0)
    proc(NBATCH - 1, 1, False, True)
    wait_scatter(1)
    plsc.subcore_barrier()

    def wout(r0, nr):
        pltpu.sync_copy(u_sp.at[pl.ds(r0, nr)], u_out.at[c, pl.ds(r0, nr)])
        pltpu.sync_copy(d_sp.at[pl.ds(r0, nr)], d_out.at[c, pl.ds(r0, nr)])

    _stripe(s, wout)


def _sc_call(src, dst, fT, elP, erP, zf, zd):
    mesh = plsc.VectorSubcoreMesh(core_axis_name="c", subcore_axis_name="s")
    return pl.kernel(
        _sc_edge,
        out_type=[jax.ShapeDtypeStruct((2, N, FW), jnp.float32),
                  jax.ShapeDtypeStruct((2, N, 16), jnp.float32)],
        mesh=mesh,
        scratch_types=[
            pltpu.VMEM((2, EB), jnp.int32),
            pltpu.VMEM((2, EB), jnp.int32),
            pltpu.VMEM((2, EB), jnp.int32),
            pltpu.VMEM((2, EB), jnp.int32),
            pltpu.VMEM((2, EB, FW), jnp.float32),
            pltpu.VMEM((2, EB, 16), jnp.float32),
            pltpu.VMEM((2, EB, 16), jnp.float32),
            pltpu.VMEM((2, EB, 16), jnp.float32),
            pltpu.VMEM_SHARED((N, FW), jnp.float32),
            pltpu.VMEM_SHARED((N, 16), jnp.float32),
            pltpu.SemaphoreType.DMA,
            pltpu.SemaphoreType.DMA,
            pltpu.SemaphoreType.DMA,
            pltpu.SemaphoreType.DMA,
        ],
        compiler_params=pltpu.CompilerParams(
            needs_layout_passes=False, use_tc_tiling_on_sc=False),
    )(src, dst, fT, elP, erP, zf, zd)


RB = 1000


def _prologue(x_ref, w_ref, al_ref, ar_ref, f_ref, el_ref, er_ref):
    c = pl.program_id(1)
    fc = jnp.dot(x_ref[...], w_ref[...], preferred_element_type=jnp.float32)
    f_ref[0] = fc
    pel = jnp.dot(fc, al_ref[...], preferred_element_type=jnp.float32)
    per = jnp.dot(fc, ar_ref[...], preferred_element_type=jnp.float32)

    @pl.when(c == 0)
    def _():
        el_ref[...] = pel
        er_ref[...] = per

    @pl.when(c == 1)
    def _():
        el_ref[...] = el_ref[...] + pel
        er_ref[...] = er_ref[...] + per


def _epilogue(u_ref, d_ref, sel_ref, bias_ref, w44_ref, b44_ref, o_ref):
    acc = jnp.zeros((RB, OUT), jnp.float32)
    for c in range(2):
        uc = u_ref[c]
        dinv = 1.0 / jnp.maximum(d_ref[c][:, 0:HC], 1e-9)
        dfull = jnp.dot(dinv, sel_ref[...],
                        preferred_element_type=jnp.float32)
        r = uc * dfull + bias_ref[c][None, :]
        r = jnp.where(r > 0.0, r, jnp.exp(r) - 1.0)
        acc = acc + jnp.dot(r, w44_ref[c], preferred_element_type=jnp.float32)
    o_ref[...] = acc + b44_ref[...]


def kernel(x, edge_index, W_fc, attn_l, attn_r, bias_gat, W44, b44):
    src = edge_index[0]
    dst = edge_index[1]

    rows = jnp.arange(H * OUT, dtype=jnp.int32)
    Al = jnp.zeros((H * OUT, H), jnp.float32).at[rows, rows // OUT].set(
        attn_l.reshape(-1))
    Ar = jnp.zeros((H * OUT, H), jnp.float32).at[rows, rows // OUT].set(
        attn_r.reshape(-1))

    fT, el, er = pl.pallas_call(
        _prologue,
        grid=(N // RB, 2),
        in_specs=[
            pl.BlockSpec((RB, D), lambda i, c: (i, 0)),
            pl.BlockSpec((D, FW), lambda i, c: (0, c)),
            pl.BlockSpec((FW, H), lambda i, c: (c, 0)),
            pl.BlockSpec((FW, H), lambda i, c: (c, 0)),
        ],
        out_specs=[
            pl.BlockSpec((1, RB, FW), lambda i, c: (c, i, 0)),
            pl.BlockSpec((RB, H), lambda i, c: (i, 0)),
            pl.BlockSpec((RB, H), lambda i, c: (i, 0)),
        ],
        out_shape=[
            jax.ShapeDtypeStruct((2, N, FW), jnp.float32),
            jax.ShapeDtypeStruct((N, H), jnp.float32),
            jax.ShapeDtypeStruct((N, H), jnp.float32),
        ],
    )(x, W_fc, Al, Ar)

    def pad16(t):
        return jnp.pad(
            t.T.reshape(2, HC, N).transpose(0, 2, 1),
            ((0, 0), (0, 0), (0, 16 - HC))).reshape(2 * N, 16)

    elP = pad16(el)
    erP = pad16(er)

    zf = jnp.zeros((N, FW), jnp.float32)
    zd = jnp.zeros((N, 16), jnp.float32)

    U, den = _sc_call(src, dst, fT.reshape(2 * N, FW), elP, erP, zf, zd)

    sel = jnp.kron(jnp.eye(HC, dtype=jnp.float32),
                   jnp.ones((1, OUT), jnp.float32))

    out = pl.pallas_call(
        _epilogue,
        grid=(N // RB,),
        in_specs=[
            pl.BlockSpec((2, RB, FW), lambda i: (0, i, 0)),
            pl.BlockSpec((2, RB, 16), lambda i: (0, i, 0)),
            pl.BlockSpec((HC, FW), lambda i: (0, 0)),
            pl.BlockSpec((2, FW), lambda i: (0, 0)),
            pl.BlockSpec((2, FW, OUT), lambda i: (0, 0, 0)),
            pl.BlockSpec((1, OUT), lambda i: (0, 0)),
        ],
        out_specs=pl.BlockSpec((RB, OUT), lambda i: (i, 0)),
        out_shape=jax.ShapeDtypeStruct((N, OUT), jnp.float32),
    )(U, den, sel, bias_gat.reshape(2, FW), W44.reshape(2, FW, OUT),
      b44.reshape(1, OUT))
    return out


# speedup vs baseline: 53.3674x; 1.0163x over previous
import jax
import jax.numpy as jnp
from jax import lax
from jax.experimental import pallas as pl
from jax.experimental.pallas import tpu as pltpu
from jax.experimental.pallas import tpu_sc as plsc

N = 10000
E = 320000
D = 128
H = 8
OUT = 32
HC = H // 2
FW = HC * OUT
EB = 80
NSUB = 16
EPT = E // NSUB
NBATCH = EPT // EB
RPT = 640


def _stripe(s, fn):
    @pl.when(s < NSUB - 1)
    def _():
        fn(pl.multiple_of(s * RPT, 8), RPT)

    @pl.when(s == NSUB - 1)
    def _():
        fn((NSUB - 1) * RPT, N - (NSUB - 1) * RPT)


def _sc_edge(src_hbm, dst_hbm, fa_hbm, fb_hbm, ela_hbm, elb_hbm,
             era_hbm, erb_hbm, zf_hbm, zd_hbm,
             u_out, d_out,
             srcb, dstb, sdst, featb, elsb, erdb, wpad,
             u_sp, d_sp, gsem0, gsem1, ssem0, ssem1, isem0, isem1):
    c = lax.axis_index("c")
    s = lax.axis_index("s")
    gsem = (gsem0, gsem1)
    ssem = (ssem0, ssem1)
    isem = (isem0, isem1)

    def zinit(r0, nr):
        pltpu.sync_copy(zf_hbm.at[pl.ds(r0, nr)], u_sp.at[pl.ds(r0, nr)])
        pltpu.sync_copy(zd_hbm.at[pl.ds(r0, nr)], d_sp.at[pl.ds(r0, nr)])

    _stripe(s, zinit)
    plsc.subcore_barrier()

    row0 = s * NBATCH

    def fire_idx(jj, b):
        pltpu.async_copy(src_hbm.at[row0 + jj], srcb.at[b], isem[b])
        pltpu.async_copy(dst_hbm.at[row0 + jj], dstb.at[b], isem[b])

    def wait_idx(b):
        pltpu.make_async_copy(src_hbm.at[0], srcb.at[b], isem[b]).wait()
        pltpu.make_async_copy(dst_hbm.at[0], dstb.at[b], isem[b]).wait()

    def fire_gathers(b):
        @pl.when(c == 0)
        def _():
            pltpu.async_copy(fa_hbm.at[srcb.at[b]], featb.at[b], gsem[b])
            pltpu.async_copy(ela_hbm.at[srcb.at[b]], elsb.at[b], gsem[b])
            pltpu.async_copy(era_hbm.at[dstb.at[b]], erdb.at[b], gsem[b])

        @pl.when(c == 1)
        def _():
            pltpu.async_copy(fb_hbm.at[srcb.at[b]], featb.at[b], gsem[b])
            pltpu.async_copy(elb_hbm.at[srcb.at[b]], elsb.at[b], gsem[b])
            pltpu.async_copy(erb_hbm.at[dstb.at[b]], erdb.at[b], gsem[b])

    def wait_gathers(b):
        pltpu.make_async_copy(fa_hbm.at[srcb.at[b]], featb.at[b],
                              gsem[b]).wait()
        pltpu.make_async_copy(ela_hbm.at[srcb.at[b]], elsb.at[b],
                              gsem[b]).wait()
        pltpu.make_async_copy(era_hbm.at[dstb.at[b]], erdb.at[b],
                              gsem[b]).wait()

    def fire_scatter(b):
        pltpu.async_copy(featb.at[b], u_sp.at[sdst.at[b]], ssem[b], add=True)
        pltpu.async_copy(wpad.at[b], d_sp.at[sdst.at[b]], ssem[b], add=True)

    def wait_scatter(b):
        pltpu.make_async_copy(featb.at[b], u_sp.at[sdst.at[b]],
                              ssem[b]).wait()
        pltpu.make_async_copy(wpad.at[b], d_sp.at[sdst.at[b]], ssem[b]).wait()

    def proc(j, b, first, last):
        b1 = 1 - b
        wait_gathers(b)

        def cpdst(g, carry2):
            sdst[b, pl.ds(g * 16, 16)] = dstb[b, pl.ds(g * 16, 16)]
            return carry2

        lax.fori_loop(0, EB // 16, cpdst, 0, unroll=5)

        def wphase(i, carry2):
            e = elsb[b, i, :] + erdb[b, i, :]
            e = jnp.where(e >= 0.0, e, 0.2 * e)
            wpad[b, i, :] = jnp.exp(e)
            return carry2

        lax.fori_loop(0, EB, wphase, 0, unroll=4)

        if not first:
            wait_scatter(b1)
        if not last:
            wait_idx(b1)
            fire_gathers(b1)
            if first:
                fire_idx(j + 2, b)
            else:
                @pl.when(j + 2 < NBATCH)
                def _():
                    fire_idx(j + 2, b)

        def scale(i, carry2):
            w = wpad[b, i, :]
            for hh in range(HC):
                wv = jnp.broadcast_to(w[hh], (16,))
                for k in range(2):
                    col = hh * OUT + k * 16
                    featb[b, i, pl.ds(col, 16)] = (
                        featb[b, i, pl.ds(col, 16)] * wv)
            return carry2

        lax.fori_loop(0, EB, scale, 0, unroll=2)
        fire_scatter(b)

    fire_idx(0, 0)
    fire_idx(1, 1)
    wait_idx(0)
    fire_gathers(0)
    proc(0, 0, True, False)

    def pair(g, carry):
        j0 = 2 * g + 1
        proc(j0, 1, False, False)
        proc(j0 + 1, 0, False, False)
        return carry

    lax.fori_loop(0, (NBATCH - 2) // 2, pair, 0)
    proc(NBATCH - 1, 1, False, True)
    wait_scatter(1)
    plsc.subcore_barrier()

    def wout(r0, nr):
        pltpu.sync_copy(u_sp.at[pl.ds(r0, nr)], u_out.at[c, pl.ds(r0, nr)])
        pltpu.sync_copy(d_sp.at[pl.ds(r0, nr)], d_out.at[c, pl.ds(r0, nr)])

    _stripe(s, wout)


def _sc_call(src2, dst2, fA, fB, elA, elB, erA, erB, zf, zd):
    mesh = plsc.VectorSubcoreMesh(core_axis_name="c", subcore_axis_name="s")
    return pl.kernel(
        _sc_edge,
        out_type=[jax.ShapeDtypeStruct((2, N, FW), jnp.float32),
                  jax.ShapeDtypeStruct((2, N, 16), jnp.float32)],
        mesh=mesh,
        scratch_types=[
            pltpu.VMEM((2, EB), jnp.int32),
            pltpu.VMEM((2, EB), jnp.int32),
            pltpu.VMEM((2, EB), jnp.int32),
            pltpu.VMEM((2, EB, FW), jnp.float32),
            pltpu.VMEM((2, EB, 16), jnp.float32),
            pltpu.VMEM((2, EB, 16), jnp.float32),
            pltpu.VMEM((2, EB, 16), jnp.float32),
            pltpu.VMEM_SHARED((N, FW), jnp.float32),
            pltpu.VMEM_SHARED((N, 16), jnp.float32),
            pltpu.SemaphoreType.DMA,
            pltpu.SemaphoreType.DMA,
            pltpu.SemaphoreType.DMA,
            pltpu.SemaphoreType.DMA,
            pltpu.SemaphoreType.DMA,
            pltpu.SemaphoreType.DMA,
        ],
        compiler_params=pltpu.CompilerParams(
            needs_layout_passes=False, use_tc_tiling_on_sc=False),
    )(src2, dst2, fA, fB, elA, elB, erA, erB, zf, zd)


RB = 1000


def _prologue(x_ref, w_ref, al_ref, ar_ref, f_ref, el_ref, er_ref):
    c = pl.program_id(1)
    fc = jnp.dot(x_ref[...], w_ref[...], preferred_element_type=jnp.float32)
    f_ref[0] = fc
    pel = jnp.dot(fc, al_ref[...], preferred_element_type=jnp.float32)
    per = jnp.dot(fc, ar_ref[...], preferred_element_type=jnp.float32)

    @pl.when(c == 0)
    def _():
        el_ref[...] = pel
        er_ref[...] = per

    @pl.when(c == 1)
    def _():
        el_ref[...] = el_ref[...] + pel
        er_ref[...] = er_ref[...] + per


def _epilogue(u_ref, d_ref, sel_ref, bias_ref, w44_ref, b44_ref, o_ref):
    acc = jnp.zeros((RB, OUT), jnp.float32)
    for c in range(2):
        uc = u_ref[c]
        dinv = 1.0 / jnp.maximum(d_ref[c][:, 0:HC], 1e-9)
        dfull = jnp.dot(dinv, sel_ref[...],
                        preferred_element_type=jnp.float32)
        r = uc * dfull + bias_ref[c][None, :]
        r = jnp.where(r > 0.0, r, jnp.exp(r) - 1.0)
        acc = acc + jnp.dot(r, w44_ref[c], preferred_element_type=jnp.float32)
    o_ref[...] = acc + b44_ref[...]


def kernel(x, edge_index, W_fc, attn_l, attn_r, bias_gat, W44, b44):
    src = edge_index[0]
    dst = edge_index[1]

    rows = jnp.arange(H * OUT, dtype=jnp.int32)
    Al = jnp.zeros((H * OUT, H), jnp.float32).at[rows, rows // OUT].set(
        attn_l.reshape(-1))
    Ar = jnp.zeros((H * OUT, H), jnp.float32).at[rows, rows // OUT].set(
        attn_r.reshape(-1))

    fT, el, er = pl.pallas_call(
        _prologue,
        grid=(N // RB, 2),
        in_specs=[
            pl.BlockSpec((RB, D), lambda i, c: (i, 0)),
            pl.BlockSpec((D, FW), lambda i, c: (0, c)),
            pl.BlockSpec((FW, H), lambda i, c: (c, 0)),
            pl.BlockSpec((FW, H), lambda i, c: (c, 0)),
        ],
        out_specs=[
            pl.BlockSpec((1, RB, FW), lambda i, c: (c, i, 0)),
            pl.BlockSpec((RB, H), lambda i, c: (i, 0)),
            pl.BlockSpec((RB, H), lambda i, c: (i, 0)),
        ],
        out_shape=[
            jax.ShapeDtypeStruct((2, N, FW), jnp.float32),
            jax.ShapeDtypeStruct((N, H), jnp.float32),
            jax.ShapeDtypeStruct((N, H), jnp.float32),
        ],
    )(x, W_fc, Al, Ar)

    def pad16(t):
        return jnp.pad(t, ((0, 0), (0, 16 - HC)))

    zf = jnp.zeros((N, FW), jnp.float32)
    zd = jnp.zeros((N, 16), jnp.float32)

    U, den = _sc_call(
        src.reshape(E // EB, EB), dst.reshape(E // EB, EB),
        fT[0], fT[1],
        pad16(el[:, :HC]), pad16(el[:, HC:]),
        pad16(er[:, :HC]), pad16(er[:, HC:]),
        zf, zd)

    sel = jnp.kron(jnp.eye(HC, dtype=jnp.float32),
                   jnp.ones((1, OUT), jnp.float32))

    out = pl.pallas_call(
        _epilogue,
        grid=(N // RB,),
        in_specs=[
            pl.BlockSpec((2, RB, FW), lambda i: (0, i, 0)),
            pl.BlockSpec((2, RB, 16), lambda i: (0, i, 0)),
            pl.BlockSpec((HC, FW), lambda i: (0, 0)),
            pl.BlockSpec((2, FW), lambda i: (0, 0)),
            pl.BlockSpec((2, FW, OUT), lambda i: (0, 0, 0)),
            pl.BlockSpec((1, OUT), lambda i: (0, 0)),
        ],
        out_specs=pl.BlockSpec((RB, OUT), lambda i: (i, 0)),
        out_shape=jax.ShapeDtypeStruct((N, OUT), jnp.float32),
    )(U, den, sel, bias_gat.reshape(2, FW), W44.reshape(2, FW, OUT),
      b44.reshape(1, OUT))
    return out


# speedup vs baseline: 74.0537x; 1.3876x over previous
import jax
import jax.numpy as jnp
from jax import lax
from jax.experimental import pallas as pl
from jax.experimental.pallas import tpu as pltpu
from jax.experimental.pallas import tpu_sc as plsc

N = 10000
E = 320000
D = 128
H = 8
OUT = 32
HC = H // 2
FW = HC * OUT
FWE = FW + 16
EB = 80
NSUB = 16
EPT = E // NSUB
NBATCH = EPT // EB
RPT = 640
NRING = 3


def _stripe(s, fn):
    @pl.when(s < NSUB - 1)
    def _():
        fn(pl.multiple_of(s * RPT, 8), RPT)

    @pl.when(s == NSUB - 1)
    def _():
        fn((NSUB - 1) * RPT, N - (NSUB - 1) * RPT)


def _sc_edge(src_hbm, dst_hbm, fe_hbm, er_hbm, zf_hbm,
             u_out,
             srcb, dstb, sdst, featb, erdb,
             u_sp, gsem0, gsem1, gsem2, ssem0, ssem1, ssem2,
             isem0, isem1, isem2):
    c = lax.axis_index("c")
    s = lax.axis_index("s")
    gsem = (gsem0, gsem1, gsem2)
    ssem = (ssem0, ssem1, ssem2)
    isem = (isem0, isem1, isem2)

    def zinit(r0, nr):
        pltpu.sync_copy(zf_hbm.at[pl.ds(r0, nr)], u_sp.at[pl.ds(r0, nr)])

    _stripe(s, zinit)
    plsc.subcore_barrier()

    row0 = s * NBATCH

    def fire_idx(jj, b):
        pltpu.async_copy(src_hbm.at[row0 + jj], srcb.at[b], isem[b])
        pltpu.async_copy(dst_hbm.at[row0 + jj], dstb.at[b], isem[b])

    def wait_idx(b):
        pltpu.make_async_copy(src_hbm.at[0], srcb.at[b], isem[b]).wait()
        pltpu.make_async_copy(dst_hbm.at[0], dstb.at[b], isem[b]).wait()

    def fire_gathers(b):
        @pl.when(c == 0)
        def _():
            pltpu.async_copy(fe_hbm.at[0].at[srcb.at[b]], featb.at[b],
                             gsem[b])
            pltpu.async_copy(er_hbm.at[0].at[dstb.at[b]], erdb.at[b],
                             gsem[b])

        @pl.when(c == 1)
        def _():
            pltpu.async_copy(fe_hbm.at[1].at[srcb.at[b]], featb.at[b],
                             gsem[b])
            pltpu.async_copy(er_hbm.at[1].at[dstb.at[b]], erdb.at[b],
                             gsem[b])

    def wait_gathers(b):
        pltpu.make_async_copy(fe_hbm.at[0].at[srcb.at[b]], featb.at[b],
                              gsem[b]).wait()
        pltpu.make_async_copy(er_hbm.at[0].at[dstb.at[b]], erdb.at[b],
                              gsem[b]).wait()

    def fire_scatter(b):
        pltpu.async_copy(featb.at[b], u_sp.at[sdst.at[b]], ssem[b], add=True)

    def wait_scatter(b):
        pltpu.make_async_copy(featb.at[b], u_sp.at[sdst.at[b]],
                              ssem[b]).wait()

    def proc(j, b, skip_ws, g2, i3):
        bn = (b + 2) % NRING
        wait_gathers(b)

        def cpdst(g, carry2):
            sdst[b, pl.ds(g * 16, 16)] = dstb[b, pl.ds(g * 16, 16)]
            return carry2

        lax.fori_loop(0, EB // 16, cpdst, 0, unroll=5)

        def wphase(i, carry2):
            e = featb[b, i, pl.ds(FW, 16)] + erdb[b, i, :]
            e = jnp.where(e >= 0.0, e, 0.2 * e)
            featb[b, i, pl.ds(FW, 16)] = jnp.exp(e)
            return carry2

        lax.fori_loop(0, EB, wphase, 0, unroll=4)

        if not skip_ws:
            wait_scatter(bn)
        if g2:
            wait_idx(bn)
            fire_gathers(bn)
        if i3 == "always":
            fire_idx(j + 3, b)
        elif i3 == "guard":
            @pl.when(j + 3 < NBATCH)
            def _():
                fire_idx(j + 3, b)

        def scale(i, carry2):
            w = featb[b, i, pl.ds(FW, 16)]
            for hh in range(HC):
                wv = jnp.broadcast_to(w[hh], (16,))
                for k in range(2):
                    col = hh * OUT + k * 16
                    featb[b, i, pl.ds(col, 16)] = (
                        featb[b, i, pl.ds(col, 16)] * wv)
            return carry2

        lax.fori_loop(0, EB, scale, 0, unroll=2)
        fire_scatter(b)

    fire_idx(0, 0)
    fire_idx(1, 1)
    fire_idx(2, 2)
    wait_idx(0)
    fire_gathers(0)
    wait_idx(1)
    fire_gathers(1)
    proc(0, 0, True, True, "always")
    proc(1, 1, False, True, "always")

    def triple(t, carry):
        j0 = 3 * t + 2
        proc(j0, 2, False, True, "guard")
        proc(j0 + 1, 0, False, True, "guard")
        proc(j0 + 2, 1, False, True, "guard")
        return carry

    lax.fori_loop(0, (NBATCH - 4) // 3, triple, 0)
    proc(NBATCH - 2, 2, False, False, "never")
    proc(NBATCH - 1, 0, False, False, "never")
    wait_scatter(0)
    plsc.subcore_barrier()

    def wout(r0, nr):
        pltpu.sync_copy(u_sp.at[pl.ds(r0, nr)], u_out.at[c, pl.ds(r0, nr)])

    _stripe(s, wout)


def _sc_call(src2, dst2, fe, er2, zf):
    mesh = plsc.VectorSubcoreMesh(core_axis_name="c", subcore_axis_name="s")
    return pl.kernel(
        _sc_edge,
        out_type=jax.ShapeDtypeStruct((2, N, FWE), jnp.float32),
        mesh=mesh,
        scratch_types=[
            pltpu.VMEM((NRING, EB), jnp.int32),
            pltpu.VMEM((NRING, EB), jnp.int32),
            pltpu.VMEM((NRING, EB), jnp.int32),
            pltpu.VMEM((NRING, EB, FWE), jnp.float32),
            pltpu.VMEM((NRING, EB, 16), jnp.float32),
            pltpu.VMEM_SHARED((N, FWE), jnp.float32),
            pltpu.SemaphoreType.DMA,
            pltpu.SemaphoreType.DMA,
            pltpu.SemaphoreType.DMA,
            pltpu.SemaphoreType.DMA,
            pltpu.SemaphoreType.DMA,
            pltpu.SemaphoreType.DMA,
            pltpu.SemaphoreType.DMA,
            pltpu.SemaphoreType.DMA,
            pltpu.SemaphoreType.DMA,
        ],
        compiler_params=pltpu.CompilerParams(
            needs_layout_passes=False, use_tc_tiling_on_sc=False),
    )(src2, dst2, fe, er2, zf)


RB = 1000


def _prologue(x_ref, w_ref, al_ref, ar_ref, fe_ref, er_ref):
    fc = jnp.dot(x_ref[...], w_ref[...], preferred_element_type=jnp.float32)
    pel = jnp.dot(fc, al_ref[0], preferred_element_type=jnp.float32)
    per = jnp.dot(fc, ar_ref[0], preferred_element_type=jnp.float32)
    z12 = jnp.zeros((RB, 12), jnp.float32)
    fe_ref[0] = jnp.concatenate([fc, pel, z12], axis=1)
    er_ref[0] = jnp.concatenate([per, z12], axis=1)


def _epilogue(u_ref, sel_ref, bias_ref, w44_ref, b44_ref, o_ref):
    acc = jnp.zeros((RB, OUT), jnp.float32)
    for c in range(2):
        uc = u_ref[c, :, 0:FW]
        dinv = 1.0 / jnp.maximum(u_ref[c, :, FW:FW + HC], 1e-9)
        dfull = jnp.dot(dinv, sel_ref[...],
                        preferred_element_type=jnp.float32)
        r = uc * dfull + bias_ref[c][None, :]
        r = jnp.where(r > 0.0, r, jnp.exp(r) - 1.0)
        acc = acc + jnp.dot(r, w44_ref[c], preferred_element_type=jnp.float32)
    o_ref[...] = acc + b44_ref[...]


def kernel(x, edge_index, W_fc, attn_l, attn_r, bias_gat, W44, b44):
    src = edge_index[0]
    dst = edge_index[1]

    rows = jnp.arange(FW, dtype=jnp.int32)
    z = jnp.zeros((2, FW, HC), jnp.float32)
    ridx = jnp.tile(rows, 2)
    cidx = jnp.repeat(jnp.arange(2, dtype=jnp.int32), FW)
    Al = z.at[cidx, ridx, ridx // OUT].set(attn_l.reshape(-1))
    Ar = z.at[cidx, ridx, ridx // OUT].set(attn_r.reshape(-1))

    fe, er2 = pl.pallas_call(
        _prologue,
        grid=(N // RB, 2),
        in_specs=[
            pl.BlockSpec((RB, D), lambda i, c: (i, 0)),
            pl.BlockSpec((D, FW), lambda i, c: (0, c)),
            pl.BlockSpec((1, FW, HC), lambda i, c: (c, 0, 0)),
            pl.BlockSpec((1, FW, HC), lambda i, c: (c, 0, 0)),
        ],
        out_specs=[
            pl.BlockSpec((1, RB, FWE), lambda i, c: (c, i, 0)),
            pl.BlockSpec((1, RB, 16), lambda i, c: (c, i, 0)),
        ],
        out_shape=[
            jax.ShapeDtypeStruct((2, N, FWE), jnp.float32),
            jax.ShapeDtypeStruct((2, N, 16), jnp.float32),
        ],
    )(x, W_fc, Al, Ar)

    zf = jnp.zeros((N, FWE), jnp.float32)

    U = _sc_call(src.reshape(E // EB, EB), dst.reshape(E // EB, EB),
                 fe, er2, zf)

    sel = jnp.kron(jnp.eye(HC, dtype=jnp.float32),
                   jnp.ones((1, OUT), jnp.float32))

    out = pl.pallas_call(
        _epilogue,
        grid=(N // RB,),
        in_specs=[
            pl.BlockSpec((2, RB, FWE), lambda i: (0, i, 0)),
            pl.BlockSpec((HC, FW), lambda i: (0, 0)),
            pl.BlockSpec((2, FW), lambda i: (0, 0)),
            pl.BlockSpec((2, FW, OUT), lambda i: (0, 0, 0)),
            pl.BlockSpec((1, OUT), lambda i: (0, 0)),
        ],
        out_specs=pl.BlockSpec((RB, OUT), lambda i: (i, 0)),
        out_shape=jax.ShapeDtypeStruct((N, OUT), jnp.float32),
    )(U, sel, bias_gat.reshape(2, FW), W44.reshape(2, FW, OUT),
      b44.reshape(1, OUT))
    return out


# speedup vs baseline: 100.3036x; 1.3545x over previous
import jax
import jax.numpy as jnp
from jax import lax
from jax.experimental import pallas as pl
from jax.experimental.pallas import tpu as pltpu
from jax.experimental.pallas import tpu_sc as plsc

N = 10000
E = 320000
D = 128
H = 8
OUT = 32
HC = H // 2
FW = HC * OUT
FWE = FW + 16
EB = 80
NSUB = 16
EPT = E // NSUB
NBATCH = EPT // EB
RPT = 640
NRING = 3


def _stripe(s, fn):
    @pl.when(s < NSUB - 1)
    def _():
        fn(pl.multiple_of(s * RPT, 8), RPT)

    @pl.when(s == NSUB - 1)
    def _():
        fn((NSUB - 1) * RPT, N - (NSUB - 1) * RPT)


def _sc_edge(src_hbm, dst_hbm, fe_hbm, er_hbm, zf_hbm,
             u_out,
             srcb, dstb, sdst, featb, erdb,
             u_sp, gsem0, gsem1, gsem2, ssem0, ssem1, ssem2,
             isem0, isem1, isem2):
    c = lax.axis_index("c")
    s = lax.axis_index("s")
    gsem = (gsem0, gsem1, gsem2)
    ssem = (ssem0, ssem1, ssem2)
    isem = (isem0, isem1, isem2)

    def zinit(r0, nr):
        pltpu.sync_copy(zf_hbm.at[pl.ds(r0, nr)], u_sp.at[pl.ds(r0, nr)])

    _stripe(s, zinit)
    plsc.subcore_barrier()

    row0 = s * NBATCH

    def fire_idx(jj, b):
        pltpu.async_copy(src_hbm.at[row0 + jj], srcb.at[b], isem[b])
        pltpu.async_copy(dst_hbm.at[row0 + jj], dstb.at[b], isem[b])

    def wait_idx(b):
        pltpu.make_async_copy(src_hbm.at[0], srcb.at[b], isem[b]).wait()
        pltpu.make_async_copy(dst_hbm.at[0], dstb.at[b], isem[b]).wait()

    def fire_gathers(b):
        @pl.when(c == 0)
        def _():
            pltpu.async_copy(fe_hbm.at[0].at[srcb.at[b]], featb.at[b],
                             gsem[b])
            pltpu.async_copy(er_hbm.at[0].at[dstb.at[b]], erdb.at[b],
                             gsem[b])

        @pl.when(c == 1)
        def _():
            pltpu.async_copy(fe_hbm.at[1].at[srcb.at[b]], featb.at[b],
                             gsem[b])
            pltpu.async_copy(er_hbm.at[1].at[dstb.at[b]], erdb.at[b],
                             gsem[b])

    def wait_gathers(b):
        pltpu.make_async_copy(fe_hbm.at[0].at[srcb.at[b]], featb.at[b],
                              gsem[b]).wait()
        pltpu.make_async_copy(er_hbm.at[0].at[dstb.at[b]], erdb.at[b],
                              gsem[b]).wait()

    def fire_scatter(b):
        pltpu.async_copy(featb.at[b], u_sp.at[sdst.at[b]], ssem[b], add=True)

    def wait_scatter(b):
        pltpu.make_async_copy(featb.at[b], u_sp.at[sdst.at[b]],
                              ssem[b]).wait()

    def proc(j, b, skip_ws, g2, i3):
        bn = (b + 2) % NRING
        wait_gathers(b)

        @plsc.parallel_loop(0, EB // 16, 1, unroll=5)
        def cpdst(g):
            sdst[b, pl.ds(g * 16, 16)] = dstb[b, pl.ds(g * 16, 16)]

        @plsc.parallel_loop(0, EB, 1, unroll=4)
        def wphase(i):
            e = featb[b, i, pl.ds(FW, 16)] + erdb[b, i, :]
            e = jnp.where(e >= 0.0, e, 0.2 * e)
            featb[b, i, pl.ds(FW, 16)] = jnp.exp(e)

        if not skip_ws:
            wait_scatter(bn)
        if g2:
            wait_idx(bn)
            fire_gathers(bn)
        if i3 == "always":
            fire_idx(j + 3, b)
        elif i3 == "guard":
            @pl.when(j + 3 < NBATCH)
            def _():
                fire_idx(j + 3, b)

        @plsc.parallel_loop(0, EB, 1, unroll=2)
        def scale(i):
            w = featb[b, i, pl.ds(FW, 16)]
            for hh in range(HC):
                wv = jnp.broadcast_to(w[hh], (16,))
                for k in range(2):
                    col = hh * OUT + k * 16
                    featb[b, i, pl.ds(col, 16)] = (
                        featb[b, i, pl.ds(col, 16)] * wv)

        fire_scatter(b)

    fire_idx(0, 0)
    fire_idx(1, 1)
    fire_idx(2, 2)
    wait_idx(0)
    fire_gathers(0)
    wait_idx(1)
    fire_gathers(1)
    proc(0, 0, True, True, "always")
    proc(1, 1, False, True, "always")

    def triple(t, carry):
        j0 = 3 * t + 2
        proc(j0, 2, False, True, "guard")
        proc(j0 + 1, 0, False, True, "guard")
        proc(j0 + 2, 1, False, True, "guard")
        return carry

    lax.fori_loop(0, (NBATCH - 4) // 3, triple, 0)
    proc(NBATCH - 2, 2, False, False, "never")
    proc(NBATCH - 1, 0, False, False, "never")
    wait_scatter(0)
    plsc.subcore_barrier()

    def wout(r0, nr):
        pltpu.sync_copy(u_sp.at[pl.ds(r0, nr)], u_out.at[c, pl.ds(r0, nr)])

    _stripe(s, wout)


def _sc_call(src2, dst2, fe, er2, zf):
    mesh = plsc.VectorSubcoreMesh(core_axis_name="c", subcore_axis_name="s")
    return pl.kernel(
        _sc_edge,
        out_type=jax.ShapeDtypeStruct((2, N, FWE), jnp.float32),
        mesh=mesh,
        scratch_types=[
            pltpu.VMEM((NRING, EB), jnp.int32),
            pltpu.VMEM((NRING, EB), jnp.int32),
            pltpu.VMEM((NRING, EB), jnp.int32),
            pltpu.VMEM((NRING, EB, FWE), jnp.float32),
            pltpu.VMEM((NRING, EB, 16), jnp.float32),
            pltpu.VMEM_SHARED((N, FWE), jnp.float32),
            pltpu.SemaphoreType.DMA,
            pltpu.SemaphoreType.DMA,
            pltpu.SemaphoreType.DMA,
            pltpu.SemaphoreType.DMA,
            pltpu.SemaphoreType.DMA,
            pltpu.SemaphoreType.DMA,
            pltpu.SemaphoreType.DMA,
            pltpu.SemaphoreType.DMA,
            pltpu.SemaphoreType.DMA,
        ],
        compiler_params=pltpu.CompilerParams(
            needs_layout_passes=False, use_tc_tiling_on_sc=False),
    )(src2, dst2, fe, er2, zf)


RB = 1000


def _prologue(x_ref, w_ref, al_ref, ar_ref, fe_ref, er_ref):
    fc = jnp.dot(x_ref[...], w_ref[...], preferred_element_type=jnp.float32)
    pel = jnp.dot(fc, al_ref[0], preferred_element_type=jnp.float32)
    per = jnp.dot(fc, ar_ref[0], preferred_element_type=jnp.float32)
    z12 = jnp.zeros((RB, 12), jnp.float32)
    fe_ref[0] = jnp.concatenate([fc, pel, z12], axis=1)
    er_ref[0] = jnp.concatenate([per, z12], axis=1)


def _epilogue(u_ref, sel_ref, bias_ref, w44_ref, b44_ref, o_ref):
    acc = jnp.zeros((RB, OUT), jnp.float32)
    for c in range(2):
        uc = u_ref[c, :, 0:FW]
        dinv = 1.0 / jnp.maximum(u_ref[c, :, FW:FW + HC], 1e-9)
        dfull = jnp.dot(dinv, sel_ref[...],
                        preferred_element_type=jnp.float32)
        r = uc * dfull + bias_ref[c][None, :]
        r = jnp.where(r > 0.0, r, jnp.exp(r) - 1.0)
        acc = acc + jnp.dot(r, w44_ref[c], preferred_element_type=jnp.float32)
    o_ref[...] = acc + b44_ref[...]


def kernel(x, edge_index, W_fc, attn_l, attn_r, bias_gat, W44, b44):
    src = edge_index[0]
    dst = edge_index[1]

    rows = jnp.arange(FW, dtype=jnp.int32)
    z = jnp.zeros((2, FW, HC), jnp.float32)
    ridx = jnp.tile(rows, 2)
    cidx = jnp.repeat(jnp.arange(2, dtype=jnp.int32), FW)
    Al = z.at[cidx, ridx, ridx // OUT].set(attn_l.reshape(-1))
    Ar = z.at[cidx, ridx, ridx // OUT].set(attn_r.reshape(-1))

    fe, er2 = pl.pallas_call(
        _prologue,
        grid=(N // RB, 2),
        in_specs=[
            pl.BlockSpec((RB, D), lambda i, c: (i, 0)),
            pl.BlockSpec((D, FW), lambda i, c: (0, c)),
            pl.BlockSpec((1, FW, HC), lambda i, c: (c, 0, 0)),
            pl.BlockSpec((1, FW, HC), lambda i, c: (c, 0, 0)),
        ],
        out_specs=[
            pl.BlockSpec((1, RB, FWE), lambda i, c: (c, i, 0)),
            pl.BlockSpec((1, RB, 16), lambda i, c: (c, i, 0)),
        ],
        out_shape=[
            jax.ShapeDtypeStruct((2, N, FWE), jnp.float32),
            jax.ShapeDtypeStruct((2, N, 16), jnp.float32),
        ],
    )(x, W_fc, Al, Ar)

    zf = jnp.zeros((N, FWE), jnp.float32)

    U = _sc_call(src.reshape(E // EB, EB), dst.reshape(E // EB, EB),
                 fe, er2, zf)

    sel = jnp.kron(jnp.eye(HC, dtype=jnp.float32),
                   jnp.ones((1, OUT), jnp.float32))

    out = pl.pallas_call(
        _epilogue,
        grid=(N // RB,),
        in_specs=[
            pl.BlockSpec((2, RB, FWE), lambda i: (0, i, 0)),
            pl.BlockSpec((HC, FW), lambda i: (0, 0)),
            pl.BlockSpec((2, FW), lambda i: (0, 0)),
            pl.BlockSpec((2, FW, OUT), lambda i: (0, 0, 0)),
            pl.BlockSpec((1, OUT), lambda i: (0, 0)),
        ],
        out_specs=pl.BlockSpec((RB, OUT), lambda i: (i, 0)),
        out_shape=jax.ShapeDtypeStruct((N, OUT), jnp.float32),
    )(U, sel, bias_gat.reshape(2, FW), W44.reshape(2, FW, OUT),
      b44.reshape(1, OUT))
    return out
